# Initial kernel scaffold; baseline (speedup 1.0000x reference)
import jax
import jax.numpy as jnp
from jax.experimental import pallas as pl


def kernel(x, edge_index, W1, b1, Wg0, bg0, Wg1, bg1, Wc1, bc1, Wc2, bc2, sc_alphas, ff_alphas):
    raise NotImplementedError("write your pallas kernel here")


# speedup vs baseline: 15.9301x; 15.9301x over previous
import functools

import jax
import jax.numpy as jnp
from jax import lax
from jax.experimental import pallas as pl
from jax.experimental.pallas import tpu as pltpu
from jax.experimental.pallas import tpu_sc as plsc

N = 10000
D = 128
ODIM = 64
TEMP = 0.5

NPAD = 10240
E = 320000
CHUNK = 128
NW = 32
EPAD = 323584
CPW = EPAD // (NW * CHUNK)
RPT = NPAD // 16

_MESH = dict(core_axis_name="c", subcore_axis_name="s")


def _sc_degree(dst_p, zeros1):
  @functools.partial(
      pl.kernel,
      out_type=(jax.ShapeDtypeStruct((NPAD,), jnp.float32),
                jax.ShapeDtypeStruct((NPAD,), jnp.float32)),
      mesh=plsc.VectorSubcoreMesh(**_MESH),
      scratch_types=[
          pltpu.VMEM((CHUNK,), jnp.int32),
          pltpu.VMEM((CHUNK,), jnp.float32),
          pltpu.VMEM_SHARED((NPAD,), jnp.float32),
      ],
  )
  def k(dst_hbm, z_hbm, out0, out1, didx, ones_v, acc):
    c = lax.axis_index("c")
    s = lax.axis_index("s")
    wid = s * 2 + c
    for i in range(CHUNK // 16):
      ones_v[pl.ds(16 * i, 16)] = jnp.ones((16,), jnp.float32)

    @pl.when(s == 0)
    def _():
      pltpu.sync_copy(z_hbm, acc)

    plsc.subcore_barrier()

    def body(j, carry):
      base = pl.multiple_of((wid * CPW + j) * CHUNK, CHUNK)
      pltpu.sync_copy(dst_hbm.at[pl.ds(base, CHUNK)], didx)
      pltpu.sync_copy(ones_v, acc.at[didx], add=True)
      return carry

    lax.fori_loop(0, CPW, body, 0)
    plsc.subcore_barrier()
    r0 = pl.multiple_of(s * RPT, 8)

    @pl.when(c == 0)
    def _():
      pltpu.sync_copy(acc.at[pl.ds(r0, RPT)], out0.at[pl.ds(r0, RPT)])

    @pl.when(c == 1)
    def _():
      pltpu.sync_copy(acc.at[pl.ds(r0, RPT)], out1.at[pl.ds(r0, RPT)])

  return k(dst_p, zeros1)


def _sc_scatter_rows(u, src_p, dst_p, zeros2):
  @functools.partial(
      pl.kernel,
      out_type=(jax.ShapeDtypeStruct((NPAD, D), jnp.float32),
                jax.ShapeDtypeStruct((NPAD, D), jnp.float32)),
      mesh=plsc.VectorSubcoreMesh(**_MESH),
      scratch_types=[
          pltpu.VMEM((CHUNK,), jnp.int32),
          pltpu.VMEM((CHUNK,), jnp.int32),
          pltpu.VMEM((CHUNK, D), jnp.float32),
          pltpu.VMEM_SHARED((NPAD, D), jnp.float32),
          pltpu.SemaphoreType.DMA,
      ],
  )
  def k(u_hbm, src_hbm, dst_hbm, z_hbm, out0, out1, sidx, didx, rows, acc,
        sem):
    c = lax.axis_index("c")
    s = lax.axis_index("s")
    wid = s * 2 + c

    @pl.when(s == 0)
    def _():
      pltpu.sync_copy(z_hbm, acc)

    plsc.subcore_barrier()

    def body(j, carry):
      base = pl.multiple_of((wid * CPW + j) * CHUNK, CHUNK)
      pltpu.sync_copy(src_hbm.at[pl.ds(base, CHUNK)], sidx)
      pltpu.sync_copy(dst_hbm.at[pl.ds(base, CHUNK)], didx)
      pltpu.async_copy(u_hbm.at[sidx], rows, sem).wait()
      pltpu.sync_copy(rows, acc.at[didx], add=True)
      return carry

    lax.fori_loop(0, CPW, body, 0)
    plsc.subcore_barrier()
    r0 = pl.multiple_of(s * RPT, 8)

    @pl.when(c == 0)
    def _():
      pltpu.sync_copy(acc.at[pl.ds(r0, RPT)], out0.at[pl.ds(r0, RPT)])

    @pl.when(c == 1)
    def _():
      pltpu.sync_copy(acc.at[pl.ds(r0, RPT)], out1.at[pl.ds(r0, RPT)])

  return k(u, src_p, dst_p, zeros2)


_R = 2000
_G = N // _R


def _softmaxes(sa, fa):
  def sm(a):
    e = jnp.exp(a / TEMP - jnp.max(a / TEMP, axis=-1, keepdims=True))
    return e / jnp.sum(e, axis=-1, keepdims=True)
  return sm(sa), sm(fa)


def _full(shape):
  return pl.BlockSpec(shape, lambda i: (0, 0))


def _rows(width):
  return pl.BlockSpec((_R, width), lambda i: (i, 0))


def _tc_pre(x, W1, b1, Wg0, bg0, deg0, deg1, sa):
  def body(x_r, w1_r, b1_r, wg_r, bg_r, d0_r, d1_r, sa_r, h_r, u_r, dv_r):
    sw, _ = _softmaxes(sa_r[...], jnp.zeros((1, 3), jnp.float32))
    s01 = sw[0:1, 1:2]
    h = jnp.dot(x_r[...], w1_r[...],
                preferred_element_type=jnp.float32) + b1_r[...]
    t = jnp.maximum(s01 * h, 0.0)
    z = jnp.dot(t, wg_r[...], preferred_element_type=jnp.float32) + bg_r[...]
    dinv = lax.rsqrt(d0_r[...] + d1_r[...] + 1.0)
    h_r[...] = h
    u_r[...] = dinv * z
    dv_r[...] = dinv

  return pl.pallas_call(
      body,
      grid=(_G,),
      in_specs=[
          _rows(D), _full((D, D)), _full((1, D)), _full((D, D)),
          _full((1, D)), _rows(1), _rows(1), _full((6, 2)),
      ],
      out_specs=[_rows(D), _rows(D), _rows(1)],
      out_shape=[
          jax.ShapeDtypeStruct((N, D), jnp.float32),
          jax.ShapeDtypeStruct((N, D), jnp.float32),
          jax.ShapeDtypeStruct((N, 1), jnp.float32),
      ],
  )(x, W1, b1, Wg0, bg0, deg0, deg1, sa)


def _tc_mid(s0a, s0b, u0, h, dinv, Wg1, bg1, sa, fa):
  def body(sa_r_, sb_r_, u0_r, h_r, dv_r, wg_r, bg_r, al_r, fl_r, f1_r, u1_r):
    sw, fw = _softmaxes(al_r[...], fl_r[...])
    dinv = dv_r[...]
    agg = dinv * (sa_r_[...] + sb_r_[...] + u0_r[...])
    f1 = jnp.where(agg > 0, agg, jnp.exp(agg) - 1.0)
    a = sw[1:2, 1:2] * h_r[...]
    b = sw[2:3, 1:2] * f1
    smv = a + b
    tmp = (fw[1:2, 0:1] * jnp.maximum(smv, 0.0)
           + fw[1:2, 1:2] * jnp.maximum(0.5 * smv, 0.0)
           + fw[1:2, 2:3] * jnp.maximum(jnp.maximum(a, b), 0.0))
    z = jnp.dot(tmp, wg_r[...], preferred_element_type=jnp.float32) + bg_r[...]
    f1_r[...] = f1
    u1_r[...] = dinv * z

  return pl.pallas_call(
      body,
      grid=(_G,),
      in_specs=[
          _rows(D), _rows(D), _rows(D), _rows(D), _rows(1),
          _full((D, D)), _full((1, D)), _full((6, 2)), _full((3, 3)),
      ],
      out_specs=[_rows(D), _rows(D)],
      out_shape=[
          jax.ShapeDtypeStruct((N, D), jnp.float32),
          jax.ShapeDtypeStruct((N, D), jnp.float32),
      ],
  )(s0a, s0b, u0, h, dinv, Wg1, bg1, sa, fa)


def _tc_post(s1a, s1b, u1, h, f1, dinv, Wc1, bc1, Wc2, bc2, sa, fa):
  def body(sa_r_, sb_r_, u1_r, h_r, f1_r, dv_r, w1_r, b1_r, w2_r, b2_r,
           al_r, fl_r, o_r):
    sw, fw = _softmaxes(al_r[...], fl_r[...])
    dinv = dv_r[...]
    agg = dinv * (sa_r_[...] + sb_r_[...] + u1_r[...])
    f2 = jnp.where(agg > 0, agg, jnp.exp(agg) - 1.0)
    a = sw[3:4, 1:2] * h_r[...]
    b = sw[4:5, 1:2] * f1_r[...]
    cc = sw[5:6, 1:2] * f2
    smv = a + b + cc
    mx = jnp.maximum(jnp.maximum(a, b), cc)
    tmp = (fw[2:3, 0:1] * jnp.maximum(smv, 0.0)
           + fw[2:3, 1:2] * jnp.maximum(smv * (1.0 / 3.0), 0.0)
           + fw[2:3, 2:3] * jnp.maximum(mx, 0.0))
    hh = jnp.maximum(
        jnp.dot(tmp, w1_r[...], preferred_element_type=jnp.float32)
        + b1_r[...], 0.0)
    o_r[...] = jnp.dot(hh, w2_r[...],
                       preferred_element_type=jnp.float32) + b2_r[...]

  return pl.pallas_call(
      body,
      grid=(_G,),
      in_specs=[
          _rows(D), _rows(D), _rows(D), _rows(D), _rows(D), _rows(1),
          _full((D, D)), _full((1, D)), _full((D, ODIM)), _full((1, ODIM)),
          _full((6, 2)), _full((3, 3)),
      ],
      out_specs=[_rows(ODIM)],
      out_shape=[jax.ShapeDtypeStruct((N, ODIM), jnp.float32)],
  )(s1a, s1b, u1, h, f1, dinv, Wc1, bc1, Wc2, bc2, sa, fa)[0]


def kernel(x, edge_index, W1, b1, Wg0, bg0, Wg1, bg1, Wc1, bc1, Wc2, bc2,
           sc_alphas, ff_alphas):
  src = edge_index[0].astype(jnp.int32)
  dst = edge_index[1].astype(jnp.int32)
  pad = jnp.arange(EPAD - E, dtype=jnp.int32)
  src_p = jnp.concatenate([src, pad % N])
  dst_p = jnp.concatenate([dst, N + pad % (NPAD - N)])

  zeros1 = jnp.zeros((NPAD,), jnp.float32)
  zeros2 = jnp.zeros((NPAD, D), jnp.float32)

  b1r = b1.reshape(1, D)
  bg0r = bg0.reshape(1, D)
  bg1r = bg1.reshape(1, D)
  bc1r = bc1.reshape(1, D)
  bc2r = bc2.reshape(1, ODIM)

  deg0, deg1 = _sc_degree(dst_p, zeros1)
  h, u0, dinv = _tc_pre(x, W1, b1r, Wg0, bg0r, deg0.reshape(NPAD, 1),
                        deg1.reshape(NPAD, 1), sc_alphas)
  s0a, s0b = _sc_scatter_rows(u0, src_p, dst_p, zeros2)
  f1, u1 = _tc_mid(s0a, s0b, u0, h, dinv, Wg1, bg1r, sc_alphas, ff_alphas)
  s1a, s1b = _sc_scatter_rows(u1, src_p, dst_p, zeros2)
  return _tc_post(s1a, s1b, u1, h, f1, dinv, Wc1, bc1r, Wc2, bc2r,
                  sc_alphas, ff_alphas)


# speedup vs baseline: 29.1123x; 1.8275x over previous
import functools

import jax
import jax.numpy as jnp
from jax import lax
from jax.experimental import pallas as pl
from jax.experimental.pallas import tpu as pltpu
from jax.experimental.pallas import tpu_sc as plsc

N = 10000
D = 128
ODIM = 64
TEMP = 0.5

NPAD = 10240
E = 320000
CHUNK = 128
NW = 32
EPAD = 327680
CPW = EPAD // (NW * CHUNK)
EPW = EPAD // NW
RPT = NPAD // 16

_MESH = dict(core_axis_name="c", subcore_axis_name="s")


def _sc_degree(ei_p, zeros1):
  @functools.partial(
      pl.kernel,
      out_type=(jax.ShapeDtypeStruct((NPAD,), jnp.float32),
                jax.ShapeDtypeStruct((NPAD,), jnp.float32)),
      mesh=plsc.VectorSubcoreMesh(**_MESH),
      scratch_types=[
          pltpu.VMEM((CPW, 2, CHUNK), jnp.int32),
          pltpu.VMEM((CHUNK,), jnp.float32),
          pltpu.VMEM_SHARED((NPAD,), jnp.float32),
          pltpu.SemaphoreType.DMA,
      ],
  )
  def k(ei_hbm, z_hbm, out0, out1, eidx, ones_v, acc, sem):
    c = lax.axis_index("c")
    s = lax.axis_index("s")
    wid = s * 2 + c
    for i in range(CHUNK // 16):
      ones_v[pl.ds(16 * i, 16)] = jnp.ones((16,), jnp.float32)
    pltpu.sync_copy(ei_hbm.at[wid], eidx)

    @pl.when(s == 0)
    def _():
      pltpu.sync_copy(z_hbm, acc)

    plsc.subcore_barrier()

    def fire(j, carry):
      pltpu.async_copy(ones_v, acc.at[eidx.at[j, 1]], sem, add=True)
      return carry

    lax.fori_loop(0, CPW, fire, 0)

    def drain(j, carry):
      pltpu.make_async_copy(ones_v, acc.at[eidx.at[j, 1]], sem).wait()
      return carry

    lax.fori_loop(0, CPW, drain, 0)
    plsc.subcore_barrier()
    r0 = pl.multiple_of(s * RPT, 8)

    @pl.when(c == 0)
    def _():
      pltpu.sync_copy(acc.at[pl.ds(r0, RPT)], out0.at[pl.ds(r0, RPT)])

    @pl.when(c == 1)
    def _():
      pltpu.sync_copy(acc.at[pl.ds(r0, RPT)], out1.at[pl.ds(r0, RPT)])

  return k(ei_p, zeros1)


def _sc_scatter_rows(u, ei_p, zeros2):
  @functools.partial(
      pl.kernel,
      out_type=(jax.ShapeDtypeStruct((NPAD, D), jnp.float32),
                jax.ShapeDtypeStruct((NPAD, D), jnp.float32)),
      mesh=plsc.VectorSubcoreMesh(**_MESH),
      scratch_types=[
          pltpu.VMEM((2, 2, CHUNK), jnp.int32),
          [pltpu.VMEM((CHUNK, D), jnp.float32)] * 2,
          [pltpu.SemaphoreType.DMA] * 2,
          [pltpu.SemaphoreType.DMA] * 2,
          pltpu.VMEM_SHARED((NPAD, D), jnp.float32),
      ],
  )
  def k(u_hbm, ei_hbm, z_hbm, out0, out1, eidx, rows, gsem, isem, acc):
    c = lax.axis_index("c")
    s = lax.axis_index("s")
    wid = s * 2 + c

    pltpu.sync_copy(ei_hbm.at[wid, 0], eidx.at[0])
    pltpu.async_copy(u_hbm.at[eidx.at[0, 0]], rows[0], gsem[0])
    pltpu.async_copy(ei_hbm.at[wid, 1], eidx.at[1], isem[1])

    @pl.when(s == 0)
    def _():
      pltpu.sync_copy(z_hbm, acc)

    plsc.subcore_barrier()

    def body(g, carry):
      for b in range(2):
        j = 2 * g + b

        @pl.when(j + 1 < CPW)
        def _():
          pltpu.make_async_copy(ei_hbm.at[wid, j + 1], eidx.at[1 - b],
                                isem[1 - b]).wait()
          pltpu.async_copy(u_hbm.at[eidx.at[1 - b, 0]], rows[1 - b],
                           gsem[1 - b])

        pltpu.make_async_copy(u_hbm.at[eidx.at[b, 0]], rows[b],
                              gsem[b]).wait()
        pltpu.sync_copy(rows[b], acc.at[eidx.at[b, 1]], add=True)

        @pl.when(j + 2 < CPW)
        def _():
          pltpu.async_copy(ei_hbm.at[wid, j + 2], eidx.at[b], isem[b])

      return carry

    lax.fori_loop(0, CPW // 2, body, 0)
    plsc.subcore_barrier()
    r0 = pl.multiple_of(s * RPT, 8)

    @pl.when(c == 0)
    def _():
      pltpu.sync_copy(acc.at[pl.ds(r0, RPT)], out0.at[pl.ds(r0, RPT)])

    @pl.when(c == 1)
    def _():
      pltpu.sync_copy(acc.at[pl.ds(r0, RPT)], out1.at[pl.ds(r0, RPT)])

  return k(u, ei_p, zeros2)


_R = 2000
_G = N // _R


def _softmaxes(sa, fa):
  def sm(a):
    e = jnp.exp(a / TEMP - jnp.max(a / TEMP, axis=-1, keepdims=True))
    return e / jnp.sum(e, axis=-1, keepdims=True)
  return sm(sa), sm(fa)


def _full(shape):
  return pl.BlockSpec(shape, lambda i: (0, 0))


def _rows(width):
  return pl.BlockSpec((_R, width), lambda i: (i, 0))


def _tc_pre(x, W1, b1, Wg0, bg0, deg0, deg1, sa):
  def body(x_r, w1_r, b1_r, wg_r, bg_r, d0_r, d1_r, sa_r, h_r, u_r, dv_r):
    sw, _ = _softmaxes(sa_r[...], jnp.zeros((1, 3), jnp.float32))
    s01 = sw[0:1, 1:2]
    h = jnp.dot(x_r[...], w1_r[...],
                preferred_element_type=jnp.float32) + b1_r[...]
    t = jnp.maximum(s01 * h, 0.0)
    z = jnp.dot(t, wg_r[...], preferred_element_type=jnp.float32) + bg_r[...]
    dinv = lax.rsqrt(d0_r[...] + d1_r[...] + 1.0)
    h_r[...] = h
    u_r[...] = dinv * z
    dv_r[...] = dinv

  return pl.pallas_call(
      body,
      grid=(_G,),
      in_specs=[
          _rows(D), _full((D, D)), _full((1, D)), _full((D, D)),
          _full((1, D)), _rows(1), _rows(1), _full((6, 2)),
      ],
      out_specs=[_rows(D), _rows(D), _rows(1)],
      out_shape=[
          jax.ShapeDtypeStruct((N, D), jnp.float32),
          jax.ShapeDtypeStruct((N, D), jnp.float32),
          jax.ShapeDtypeStruct((N, 1), jnp.float32),
      ],
  )(x, W1, b1, Wg0, bg0, deg0, deg1, sa)


def _tc_mid(s0a, s0b, u0, h, dinv, Wg1, bg1, sa, fa):
  def body(sa_r_, sb_r_, u0_r, h_r, dv_r, wg_r, bg_r, al_r, fl_r, f1_r, u1_r):
    sw, fw = _softmaxes(al_r[...], fl_r[...])
    dinv = dv_r[...]
    agg = dinv * (sa_r_[...] + sb_r_[...] + u0_r[...])
    f1 = jnp.where(agg > 0, agg, jnp.exp(agg) - 1.0)
    a = sw[1:2, 1:2] * h_r[...]
    b = sw[2:3, 1:2] * f1
    smv = a + b
    tmp = (fw[1:2, 0:1] * jnp.maximum(smv, 0.0)
           + fw[1:2, 1:2] * jnp.maximum(0.5 * smv, 0.0)
           + fw[1:2, 2:3] * jnp.maximum(jnp.maximum(a, b), 0.0))
    z = jnp.dot(tmp, wg_r[...], preferred_element_type=jnp.float32) + bg_r[...]
    f1_r[...] = f1
    u1_r[...] = dinv * z

  return pl.pallas_call(
      body,
      grid=(_G,),
      in_specs=[
          _rows(D), _rows(D), _rows(D), _rows(D), _rows(1),
          _full((D, D)), _full((1, D)), _full((6, 2)), _full((3, 3)),
      ],
      out_specs=[_rows(D), _rows(D)],
      out_shape=[
          jax.ShapeDtypeStruct((N, D), jnp.float32),
          jax.ShapeDtypeStruct((N, D), jnp.float32),
      ],
  )(s0a, s0b, u0, h, dinv, Wg1, bg1, sa, fa)


def _tc_post(s1a, s1b, u1, h, f1, dinv, Wc1, bc1, Wc2, bc2, sa, fa):
  def body(sa_r_, sb_r_, u1_r, h_r, f1_r, dv_r, w1_r, b1_r, w2_r, b2_r,
           al_r, fl_r, o_r):
    sw, fw = _softmaxes(al_r[...], fl_r[...])
    dinv = dv_r[...]
    agg = dinv * (sa_r_[...] + sb_r_[...] + u1_r[...])
    f2 = jnp.where(agg > 0, agg, jnp.exp(agg) - 1.0)
    a = sw[3:4, 1:2] * h_r[...]
    b = sw[4:5, 1:2] * f1_r[...]
    cc = sw[5:6, 1:2] * f2
    smv = a + b + cc
    mx = jnp.maximum(jnp.maximum(a, b), cc)
    tmp = (fw[2:3, 0:1] * jnp.maximum(smv, 0.0)
           + fw[2:3, 1:2] * jnp.maximum(smv * (1.0 / 3.0), 0.0)
           + fw[2:3, 2:3] * jnp.maximum(mx, 0.0))
    hh = jnp.maximum(
        jnp.dot(tmp, w1_r[...], preferred_element_type=jnp.float32)
        + b1_r[...], 0.0)
    o_r[...] = jnp.dot(hh, w2_r[...],
                       preferred_element_type=jnp.float32) + b2_r[...]

  return pl.pallas_call(
      body,
      grid=(_G,),
      in_specs=[
          _rows(D), _rows(D), _rows(D), _rows(D), _rows(D), _rows(1),
          _full((D, D)), _full((1, D)), _full((D, ODIM)), _full((1, ODIM)),
          _full((6, 2)), _full((3, 3)),
      ],
      out_specs=[_rows(ODIM)],
      out_shape=[jax.ShapeDtypeStruct((N, ODIM), jnp.float32)],
  )(s1a, s1b, u1, h, f1, dinv, Wc1, bc1, Wc2, bc2, sa, fa)[0]


def kernel(x, edge_index, W1, b1, Wg0, bg0, Wg1, bg1, Wc1, bc1, Wc2, bc2,
           sc_alphas, ff_alphas):
  src = edge_index[0].astype(jnp.int32)
  dst = edge_index[1].astype(jnp.int32)
  pad = jnp.arange(EPAD - E, dtype=jnp.int32)
  src_p = jnp.concatenate([src, pad % N]).reshape(NW, CPW, CHUNK)
  dst_p = jnp.concatenate([dst, N + pad % (NPAD - N)]).reshape(NW, CPW, CHUNK)
  ei_p = jnp.stack([src_p, dst_p], axis=2)

  zeros1 = jnp.zeros((NPAD,), jnp.float32)
  zeros2 = jnp.zeros((NPAD, D), jnp.float32)

  b1r = b1.reshape(1, D)
  bg0r = bg0.reshape(1, D)
  bg1r = bg1.reshape(1, D)
  bc1r = bc1.reshape(1, D)
  bc2r = bc2.reshape(1, ODIM)

  deg0, deg1 = _sc_degree(ei_p, zeros1)
  h, u0, dinv = _tc_pre(x, W1, b1r, Wg0, bg0r, deg0.reshape(NPAD, 1),
                        deg1.reshape(NPAD, 1), sc_alphas)
  s0a, s0b = _sc_scatter_rows(u0, ei_p, zeros2)
  f1, u1 = _tc_mid(s0a, s0b, u0, h, dinv, Wg1, bg1r, sc_alphas, ff_alphas)
  s1a, s1b = _sc_scatter_rows(u1, ei_p, zeros2)
  return _tc_post(s1a, s1b, u1, h, f1, dinv, Wc1, bc1r, Wc2, bc2r,
                  sc_alphas, ff_alphas)


# speedup vs baseline: 32.0420x; 1.1006x over previous
import functools

import jax
import jax.numpy as jnp
from jax import lax
from jax.experimental import pallas as pl
from jax.experimental.pallas import tpu as pltpu
from jax.experimental.pallas import tpu_sc as plsc

N = 10000
D = 128
ODIM = 64
TEMP = 0.5

NPAD = 10240
E = 320000
CHUNK = 128
NW = 32
EPAD = 327680
CPW = EPAD // (NW * CHUNK)
EPW = EPAD // NW
RPT = NPAD // 16

_MESH = dict(core_axis_name="c", subcore_axis_name="s")


def _sc_degree(ei_p, zeros1):
  @functools.partial(
      pl.kernel,
      out_type=(jax.ShapeDtypeStruct((NPAD,), jnp.float32),
                jax.ShapeDtypeStruct((NPAD,), jnp.float32)),
      mesh=plsc.VectorSubcoreMesh(**_MESH),
      scratch_types=[
          pltpu.VMEM((CPW, 2, CHUNK), jnp.int32),
          pltpu.VMEM((CHUNK,), jnp.float32),
          pltpu.VMEM_SHARED((NPAD,), jnp.float32),
          pltpu.SemaphoreType.DMA,
      ],
  )
  def k(ei_hbm, z_hbm, out0, out1, eidx, ones_v, acc, sem):
    c = lax.axis_index("c")
    s = lax.axis_index("s")
    wid = s * 2 + c
    for i in range(CHUNK // 16):
      ones_v[pl.ds(16 * i, 16)] = jnp.ones((16,), jnp.float32)
    pltpu.sync_copy(ei_hbm.at[wid], eidx)

    @pl.when(s == 0)
    def _():
      pltpu.sync_copy(z_hbm, acc)

    plsc.subcore_barrier()

    def fire(j, carry):
      pltpu.async_copy(ones_v, acc.at[eidx.at[j, 1]], sem, add=True)
      return carry

    lax.fori_loop(0, CPW, fire, 0)

    def drain(j, carry):
      pltpu.make_async_copy(ones_v, acc.at[eidx.at[j, 1]], sem).wait()
      return carry

    lax.fori_loop(0, CPW, drain, 0)
    plsc.subcore_barrier()
    r0 = pl.multiple_of(s * RPT, 8)

    @pl.when(c == 0)
    def _():
      pltpu.sync_copy(acc.at[pl.ds(r0, RPT)], out0.at[pl.ds(r0, RPT)])

    @pl.when(c == 1)
    def _():
      pltpu.sync_copy(acc.at[pl.ds(r0, RPT)], out1.at[pl.ds(r0, RPT)])

  return k(ei_p, zeros1)


def _sc_scatter_rows(u, ei_p, zeros2):
  @functools.partial(
      pl.kernel,
      out_type=(jax.ShapeDtypeStruct((NPAD, D), jnp.float32),
                jax.ShapeDtypeStruct((NPAD, D), jnp.float32)),
      mesh=plsc.VectorSubcoreMesh(**_MESH),
      scratch_types=[
          pltpu.VMEM((2, 2, CHUNK), jnp.int32),
          pltpu.VMEM((2, CHUNK), jnp.int32),
          [pltpu.VMEM((CHUNK, D), jnp.float32)] * 2,
          [pltpu.SemaphoreType.DMA] * 2,
          [pltpu.SemaphoreType.DMA] * 2,
          [pltpu.SemaphoreType.DMA] * 2,
          pltpu.VMEM_SHARED((NPAD, D), jnp.float32),
      ],
  )
  def k(u_hbm, ei_hbm, z_hbm, out0, out1, eidx, didx, rows, gsem, isem,
        ssem, acc):
    c = lax.axis_index("c")
    s = lax.axis_index("s")
    wid = s * 2 + c

    pltpu.sync_copy(ei_hbm.at[wid, 0], eidx.at[0])
    pltpu.async_copy(u_hbm.at[eidx.at[0, 0]], rows[0], gsem[0])
    pltpu.async_copy(ei_hbm.at[wid, 1], eidx.at[1], isem[1])

    @pl.when(s == 0)
    def _():
      pltpu.sync_copy(z_hbm, acc)

    plsc.subcore_barrier()

    def body(g, carry):
      for b in range(2):
        j = 2 * g + b

        @pl.when(j + 1 < CPW)
        def _():
          pltpu.make_async_copy(ei_hbm.at[wid, j + 1], eidx.at[1 - b],
                                isem[1 - b]).wait()

          @pl.when(j >= 1)
          def _():
            pltpu.make_async_copy(rows[1 - b], acc.at[didx.at[1 - b]],
                                  ssem[1 - b]).wait()

          pltpu.async_copy(u_hbm.at[eidx.at[1 - b, 0]], rows[1 - b],
                           gsem[1 - b])

        pltpu.make_async_copy(u_hbm.at[eidx.at[b, 0]], rows[b],
                              gsem[b]).wait()
        for i in range(CHUNK // 16):
          didx[b, pl.ds(16 * i, 16)] = eidx[b, 1, pl.ds(16 * i, 16)]
        pltpu.async_copy(rows[b], acc.at[didx.at[b]], ssem[b], add=True)

        @pl.when(j + 2 < CPW)
        def _():
          pltpu.async_copy(ei_hbm.at[wid, j + 2], eidx.at[b], isem[b])

      return carry

    lax.fori_loop(0, CPW // 2, body, 0)
    for b in range(2):
      pltpu.make_async_copy(rows[b], acc.at[didx.at[b]], ssem[b]).wait()
    plsc.subcore_barrier()
    r0 = pl.multiple_of(s * RPT, 8)

    @pl.when(c == 0)
    def _():
      pltpu.sync_copy(acc.at[pl.ds(r0, RPT)], out0.at[pl.ds(r0, RPT)])

    @pl.when(c == 1)
    def _():
      pltpu.sync_copy(acc.at[pl.ds(r0, RPT)], out1.at[pl.ds(r0, RPT)])

  return k(u, ei_p, zeros2)


_R = 2000
_G = N // _R


def _softmaxes(sa, fa):
  def sm(a):
    e = jnp.exp(a / TEMP - jnp.max(a / TEMP, axis=-1, keepdims=True))
    return e / jnp.sum(e, axis=-1, keepdims=True)
  return sm(sa), sm(fa)


def _full(shape):
  return pl.BlockSpec(shape, lambda i: (0, 0))


def _rows(width):
  return pl.BlockSpec((_R, width), lambda i: (i, 0))


def _tc_pre(x, W1, b1, Wg0, bg0, deg0, deg1, sa):
  def body(x_r, w1_r, b1_r, wg_r, bg_r, d0_r, d1_r, sa_r, h_r, u_r, dv_r):
    sw, _ = _softmaxes(sa_r[...], jnp.zeros((1, 3), jnp.float32))
    s01 = sw[0:1, 1:2]
    h = jnp.dot(x_r[...], w1_r[...],
                preferred_element_type=jnp.float32) + b1_r[...]
    t = jnp.maximum(s01 * h, 0.0)
    z = jnp.dot(t, wg_r[...], preferred_element_type=jnp.float32) + bg_r[...]
    dinv = lax.rsqrt(d0_r[...] + d1_r[...] + 1.0)
    h_r[...] = h
    u_r[...] = dinv * z
    dv_r[...] = dinv

  return pl.pallas_call(
      body,
      grid=(_G,),
      in_specs=[
          _rows(D), _full((D, D)), _full((1, D)), _full((D, D)),
          _full((1, D)), _rows(1), _rows(1), _full((6, 2)),
      ],
      out_specs=[_rows(D), _rows(D), _rows(1)],
      out_shape=[
          jax.ShapeDtypeStruct((N, D), jnp.float32),
          jax.ShapeDtypeStruct((N, D), jnp.float32),
          jax.ShapeDtypeStruct((N, 1), jnp.float32),
      ],
  )(x, W1, b1, Wg0, bg0, deg0, deg1, sa)


def _tc_mid(s0a, s0b, u0, h, dinv, Wg1, bg1, sa, fa):
  def body(sa_r_, sb_r_, u0_r, h_r, dv_r, wg_r, bg_r, al_r, fl_r, f1_r, u1_r):
    sw, fw = _softmaxes(al_r[...], fl_r[...])
    dinv = dv_r[...]
    agg = dinv * (sa_r_[...] + sb_r_[...] + u0_r[...])
    f1 = jnp.where(agg > 0, agg, jnp.exp(agg) - 1.0)
    a = sw[1:2, 1:2] * h_r[...]
    b = sw[2:3, 1:2] * f1
    smv = a + b
    tmp = (fw[1:2, 0:1] * jnp.maximum(smv, 0.0)
           + fw[1:2, 1:2] * jnp.maximum(0.5 * smv, 0.0)
           + fw[1:2, 2:3] * jnp.maximum(jnp.maximum(a, b), 0.0))
    z = jnp.dot(tmp, wg_r[...], preferred_element_type=jnp.float32) + bg_r[...]
    f1_r[...] = f1
    u1_r[...] = dinv * z

  return pl.pallas_call(
      body,
      grid=(_G,),
      in_specs=[
          _rows(D), _rows(D), _rows(D), _rows(D), _rows(1),
          _full((D, D)), _full((1, D)), _full((6, 2)), _full((3, 3)),
      ],
      out_specs=[_rows(D), _rows(D)],
      out_shape=[
          jax.ShapeDtypeStruct((N, D), jnp.float32),
          jax.ShapeDtypeStruct((N, D), jnp.float32),
      ],
  )(s0a, s0b, u0, h, dinv, Wg1, bg1, sa, fa)


def _tc_post(s1a, s1b, u1, h, f1, dinv, Wc1, bc1, Wc2, bc2, sa, fa):
  def body(sa_r_, sb_r_, u1_r, h_r, f1_r, dv_r, w1_r, b1_r, w2_r, b2_r,
           al_r, fl_r, o_r):
    sw, fw = _softmaxes(al_r[...], fl_r[...])
    dinv = dv_r[...]
    agg = dinv * (sa_r_[...] + sb_r_[...] + u1_r[...])
    f2 = jnp.where(agg > 0, agg, jnp.exp(agg) - 1.0)
    a = sw[3:4, 1:2] * h_r[...]
    b = sw[4:5, 1:2] * f1_r[...]
    cc = sw[5:6, 1:2] * f2
    smv = a + b + cc
    mx = jnp.maximum(jnp.maximum(a, b), cc)
    tmp = (fw[2:3, 0:1] * jnp.maximum(smv, 0.0)
           + fw[2:3, 1:2] * jnp.maximum(smv * (1.0 / 3.0), 0.0)
           + fw[2:3, 2:3] * jnp.maximum(mx, 0.0))
    hh = jnp.maximum(
        jnp.dot(tmp, w1_r[...], preferred_element_type=jnp.float32)
        + b1_r[...], 0.0)
    o_r[...] = jnp.dot(hh, w2_r[...],
                       preferred_element_type=jnp.float32) + b2_r[...]

  return pl.pallas_call(
      body,
      grid=(_G,),
      in_specs=[
          _rows(D), _rows(D), _rows(D), _rows(D), _rows(D), _rows(1),
          _full((D, D)), _full((1, D)), _full((D, ODIM)), _full((1, ODIM)),
          _full((6, 2)), _full((3, 3)),
      ],
      out_specs=[_rows(ODIM)],
      out_shape=[jax.ShapeDtypeStruct((N, ODIM), jnp.float32)],
  )(s1a, s1b, u1, h, f1, dinv, Wc1, bc1, Wc2, bc2, sa, fa)[0]


def kernel(x, edge_index, W1, b1, Wg0, bg0, Wg1, bg1, Wc1, bc1, Wc2, bc2,
           sc_alphas, ff_alphas):
  src = edge_index[0].astype(jnp.int32)
  dst = edge_index[1].astype(jnp.int32)
  pad = jnp.arange(EPAD - E, dtype=jnp.int32)
  src_p = jnp.concatenate([src, pad % N]).reshape(NW, CPW, CHUNK)
  dst_p = jnp.concatenate([dst, N + pad % (NPAD - N)]).reshape(NW, CPW, CHUNK)
  ei_p = jnp.stack([src_p, dst_p], axis=2)

  zeros1 = jnp.zeros((NPAD,), jnp.float32)
  zeros2 = jnp.zeros((NPAD, D), jnp.float32)

  b1r = b1.reshape(1, D)
  bg0r = bg0.reshape(1, D)
  bg1r = bg1.reshape(1, D)
  bc1r = bc1.reshape(1, D)
  bc2r = bc2.reshape(1, ODIM)

  deg0, deg1 = _sc_degree(ei_p, zeros1)
  h, u0, dinv = _tc_pre(x, W1, b1r, Wg0, bg0r, deg0.reshape(NPAD, 1),
                        deg1.reshape(NPAD, 1), sc_alphas)
  s0a, s0b = _sc_scatter_rows(u0, ei_p, zeros2)
  f1, u1 = _tc_mid(s0a, s0b, u0, h, dinv, Wg1, bg1r, sc_alphas, ff_alphas)
  s1a, s1b = _sc_scatter_rows(u1, ei_p, zeros2)
  return _tc_post(s1a, s1b, u1, h, f1, dinv, Wc1, bc1r, Wc2, bc2r,
                  sc_alphas, ff_alphas)


# speedup vs baseline: 34.6431x; 1.0812x over previous
import functools

import jax
import jax.numpy as jnp
from jax import lax
from jax.experimental import pallas as pl
from jax.experimental.pallas import tpu as pltpu
from jax.experimental.pallas import tpu_sc as plsc

N = 10000
D = 128
ODIM = 64
TEMP = 0.5

NPAD = 10032
DPAD = 10240
E = 320000
CHUNK = 128
NW = 32
EPAD = 327680
CPW = EPAD // (NW * CHUNK)
EPW = EPAD // NW
RPT = 640
RPT_LAST = NPAD - 15 * RPT

_MESH = dict(core_axis_name="c", subcore_axis_name="s")


def _writeback(acc, out0, out1, c, s, last):
  def wb(out):
    r0 = pl.multiple_of(s * RPT, 8)

    @pl.when(s < 15)
    def _():
      pltpu.sync_copy(acc.at[pl.ds(r0, RPT)], out.at[pl.ds(r0, RPT)])

    @pl.when(s == 15)
    def _():
      pltpu.sync_copy(acc.at[pl.ds(15 * RPT, last)],
                      out.at[pl.ds(15 * RPT, last)])

  @pl.when(c == 0)
  def _():
    wb(out0)

  @pl.when(c == 1)
  def _():
    wb(out1)


def _sc_degree(ei_p, zeros1):
  @functools.partial(
      pl.kernel,
      out_type=(jax.ShapeDtypeStruct((DPAD,), jnp.float32),
                jax.ShapeDtypeStruct((DPAD,), jnp.float32)),
      mesh=plsc.VectorSubcoreMesh(**_MESH),
      scratch_types=[
          pltpu.VMEM((CPW, 2, CHUNK), jnp.int32),
          pltpu.VMEM((CHUNK,), jnp.float32),
          pltpu.VMEM_SHARED((DPAD,), jnp.float32),
          pltpu.SemaphoreType.DMA,
      ],
  )
  def k(ei_hbm, z_hbm, out0, out1, eidx, ones_v, acc, sem):
    c = lax.axis_index("c")
    s = lax.axis_index("s")
    wid = s * 2 + c
    for i in range(CHUNK // 16):
      ones_v[pl.ds(16 * i, 16)] = jnp.ones((16,), jnp.float32)
    pltpu.sync_copy(ei_hbm.at[wid], eidx)

    @pl.when(s == 0)
    def _():
      pltpu.sync_copy(z_hbm, acc)

    plsc.subcore_barrier()

    def fire(j, carry):
      pltpu.async_copy(ones_v, acc.at[eidx.at[j, 1]], sem, add=True)
      return carry

    lax.fori_loop(0, CPW, fire, 0)

    def drain(j, carry):
      pltpu.make_async_copy(ones_v, acc.at[eidx.at[j, 1]], sem).wait()
      return carry

    lax.fori_loop(0, CPW, drain, 0)
    plsc.subcore_barrier()
    _writeback(acc, out0, out1, c, s, RPT)

  return k(ei_p, zeros1)


def _sc_scatter_rows(u, ei_p, zeros2):
  @functools.partial(
      pl.kernel,
      out_type=(jax.ShapeDtypeStruct((NPAD, D), jnp.float32),
                jax.ShapeDtypeStruct((NPAD, D), jnp.float32)),
      mesh=plsc.VectorSubcoreMesh(**_MESH),
      scratch_types=[
          pltpu.VMEM((6, 2, CHUNK), jnp.int32),
          [pltpu.VMEM((CHUNK, D), jnp.float32)] * 3,
          [pltpu.SemaphoreType.DMA] * 3,
          [pltpu.SemaphoreType.DMA] * 6,
          [pltpu.SemaphoreType.DMA] * 3,
          pltpu.VMEM_SHARED((NPAD, D), jnp.float32),
      ],
  )
  def k(u_hbm, ei_hbm, z_hbm, out0, out1, eidx, rows, gsem, isem, ssem,
        acc):
    c = lax.axis_index("c")
    s = lax.axis_index("s")
    wid = s * 2 + c

    def step(j, kk, when):
      def prefetch():
        pltpu.make_async_copy(ei_hbm.at[wid, j + 2], eidx.at[(kk + 2) % 6],
                              isem[(kk + 2) % 6]).wait()

        def free_rows():
          pltpu.make_async_copy(rows[(kk + 2) % 3],
                                acc.at[eidx.at[(kk + 5) % 6, 1]],
                                ssem[(kk + 2) % 3]).wait()

        when(j >= 1, free_rows)
        pltpu.async_copy(u_hbm.at[eidx.at[(kk + 2) % 6, 0]],
                         rows[(kk + 2) % 3], gsem[(kk + 2) % 3])

      when(j + 2 < CPW, prefetch)
      pltpu.make_async_copy(u_hbm.at[eidx.at[kk, 0]], rows[kk % 3],
                            gsem[kk % 3]).wait()
      pltpu.async_copy(rows[kk % 3], acc.at[eidx.at[kk, 1]],
                       ssem[kk % 3], add=True)
      def load_ahead():
        pltpu.async_copy(ei_hbm.at[wid, j + 4], eidx.at[(kk + 4) % 6],
                         isem[(kk + 4) % 6])

      when(j + 4 < CPW, load_ahead)

    pltpu.sync_copy(ei_hbm.at[wid, 0], eidx.at[0])
    pltpu.async_copy(u_hbm.at[eidx.at[0, 0]], rows[0], gsem[0])
    for q in range(1, 4):
      pltpu.async_copy(ei_hbm.at[wid, q], eidx.at[q], isem[q])
    pltpu.make_async_copy(ei_hbm.at[wid, 1], eidx.at[1], isem[1]).wait()
    pltpu.async_copy(u_hbm.at[eidx.at[1, 0]], rows[1], gsem[1])

    @pl.when(s == 0)
    def _():
      pltpu.sync_copy(z_hbm, acc)

    plsc.subcore_barrier()

    def twhen(cond, f):
      pl.when(cond)(f)

    def body(g, carry):
      for kk in range(6):
        step(6 * g + kk, kk, twhen)
      return carry

    lax.fori_loop(0, (CPW - 2) // 6, body, 0)

    def swhen(cond, f):
      if cond:
        f()

    for j in range(CPW - 2, CPW):
      step(j, j % 6, swhen)
    for j in range(CPW - 3, CPW):
      pltpu.make_async_copy(rows[j % 3], acc.at[eidx.at[j % 6, 1]],
                            ssem[j % 3]).wait()
    plsc.subcore_barrier()
    _writeback(acc, out0, out1, c, s, RPT_LAST)

  return k(u, ei_p, zeros2)


_R = 2000
_G = N // _R


def _softmaxes(sa, fa):
  def sm(a):
    e = jnp.exp(a / TEMP - jnp.max(a / TEMP, axis=-1, keepdims=True))
    return e / jnp.sum(e, axis=-1, keepdims=True)
  return sm(sa), sm(fa)


def _full(shape):
  return pl.BlockSpec(shape, lambda i: (0, 0))


def _rows(width):
  return pl.BlockSpec((_R, width), lambda i: (i, 0))


def _tc_pre(x, W1, b1, Wg0, bg0, deg0, deg1, sa):
  def body(x_r, w1_r, b1_r, wg_r, bg_r, d0_r, d1_r, sa_r, h_r, u_r, dv_r):
    sw, _ = _softmaxes(sa_r[...], jnp.zeros((1, 3), jnp.float32))
    s01 = sw[0:1, 1:2]
    h = jnp.dot(x_r[...], w1_r[...],
                preferred_element_type=jnp.float32) + b1_r[...]
    t = jnp.maximum(s01 * h, 0.0)
    z = jnp.dot(t, wg_r[...], preferred_element_type=jnp.float32) + bg_r[...]
    dinv = lax.rsqrt(d0_r[...] + d1_r[...] + 1.0)
    h_r[...] = h
    u_r[...] = dinv * z
    dv_r[...] = dinv

  return pl.pallas_call(
      body,
      grid=(_G,),
      in_specs=[
          _rows(D), _full((D, D)), _full((1, D)), _full((D, D)),
          _full((1, D)), _rows(1), _rows(1), _full((6, 2)),
      ],
      out_specs=[_rows(D), _rows(D), _rows(1)],
      out_shape=[
          jax.ShapeDtypeStruct((N, D), jnp.float32),
          jax.ShapeDtypeStruct((N, D), jnp.float32),
          jax.ShapeDtypeStruct((N, 1), jnp.float32),
      ],
  )(x, W1, b1, Wg0, bg0, deg0, deg1, sa)


def _tc_mid(s0a, s0b, u0, h, dinv, Wg1, bg1, sa, fa):
  def body(sa_r_, sb_r_, u0_r, h_r, dv_r, wg_r, bg_r, al_r, fl_r, f1_r, u1_r):
    sw, fw = _softmaxes(al_r[...], fl_r[...])
    dinv = dv_r[...]
    agg = dinv * (sa_r_[...] + sb_r_[...] + u0_r[...])
    f1 = jnp.where(agg > 0, agg, jnp.exp(agg) - 1.0)
    a = sw[1:2, 1:2] * h_r[...]
    b = sw[2:3, 1:2] * f1
    smv = a + b
    tmp = (fw[1:2, 0:1] * jnp.maximum(smv, 0.0)
           + fw[1:2, 1:2] * jnp.maximum(0.5 * smv, 0.0)
           + fw[1:2, 2:3] * jnp.maximum(jnp.maximum(a, b), 0.0))
    z = jnp.dot(tmp, wg_r[...], preferred_element_type=jnp.float32) + bg_r[...]
    f1_r[...] = f1
    u1_r[...] = dinv * z

  return pl.pallas_call(
      body,
      grid=(_G,),
      in_specs=[
          _rows(D), _rows(D), _rows(D), _rows(D), _rows(1),
          _full((D, D)), _full((1, D)), _full((6, 2)), _full((3, 3)),
      ],
      out_specs=[_rows(D), _rows(D)],
      out_shape=[
          jax.ShapeDtypeStruct((N, D), jnp.float32),
          jax.ShapeDtypeStruct((N, D), jnp.float32),
      ],
  )(s0a, s0b, u0, h, dinv, Wg1, bg1, sa, fa)


def _tc_post(s1a, s1b, u1, h, f1, dinv, Wc1, bc1, Wc2, bc2, sa, fa):
  def body(sa_r_, sb_r_, u1_r, h_r, f1_r, dv_r, w1_r, b1_r, w2_r, b2_r,
           al_r, fl_r, o_r):
    sw, fw = _softmaxes(al_r[...], fl_r[...])
    dinv = dv_r[...]
    agg = dinv * (sa_r_[...] + sb_r_[...] + u1_r[...])
    f2 = jnp.where(agg > 0, agg, jnp.exp(agg) - 1.0)
    a = sw[3:4, 1:2] * h_r[...]
    b = sw[4:5, 1:2] * f1_r[...]
    cc = sw[5:6, 1:2] * f2
    smv = a + b + cc
    mx = jnp.maximum(jnp.maximum(a, b), cc)
    tmp = (fw[2:3, 0:1] * jnp.maximum(smv, 0.0)
           + fw[2:3, 1:2] * jnp.maximum(smv * (1.0 / 3.0), 0.0)
           + fw[2:3, 2:3] * jnp.maximum(mx, 0.0))
    hh = jnp.maximum(
        jnp.dot(tmp, w1_r[...], preferred_element_type=jnp.float32)
        + b1_r[...], 0.0)
    o_r[...] = jnp.dot(hh, w2_r[...],
                       preferred_element_type=jnp.float32) + b2_r[...]

  return pl.pallas_call(
      body,
      grid=(_G,),
      in_specs=[
          _rows(D), _rows(D), _rows(D), _rows(D), _rows(D), _rows(1),
          _full((D, D)), _full((1, D)), _full((D, ODIM)), _full((1, ODIM)),
          _full((6, 2)), _full((3, 3)),
      ],
      out_specs=[_rows(ODIM)],
      out_shape=[jax.ShapeDtypeStruct((N, ODIM), jnp.float32)],
  )(s1a, s1b, u1, h, f1, dinv, Wc1, bc1, Wc2, bc2, sa, fa)[0]


def kernel(x, edge_index, W1, b1, Wg0, bg0, Wg1, bg1, Wc1, bc1, Wc2, bc2,
           sc_alphas, ff_alphas):
  src = edge_index[0].astype(jnp.int32)
  dst = edge_index[1].astype(jnp.int32)
  ppw = EPW - E // NW
  pad = jnp.arange(ppw, dtype=jnp.int32)
  wofs = jnp.arange(NW, dtype=jnp.int32)[:, None] * 997
  psrc = (pad[None, :] + wofs) % N
  pdst = N + (pad[None, :] + wofs) % (NPAD - N)
  src_p = jnp.concatenate([src.reshape(NW, E // NW), psrc],
                          axis=1).reshape(NW, CPW, CHUNK)
  dst_p = jnp.concatenate([dst.reshape(NW, E // NW), pdst],
                          axis=1).reshape(NW, CPW, CHUNK)
  ei_p = jnp.stack([src_p, dst_p], axis=2)

  zeros1 = jnp.zeros((DPAD,), jnp.float32)
  zeros2 = jnp.zeros((NPAD, D), jnp.float32)

  b1r = b1.reshape(1, D)
  bg0r = bg0.reshape(1, D)
  bg1r = bg1.reshape(1, D)
  bc1r = bc1.reshape(1, D)
  bc2r = bc2.reshape(1, ODIM)

  deg0, deg1 = _sc_degree(ei_p, zeros1)
  h, u0, dinv = _tc_pre(x, W1, b1r, Wg0, bg0r, deg0.reshape(DPAD, 1),
                        deg1.reshape(DPAD, 1), sc_alphas)
  s0a, s0b = _sc_scatter_rows(u0, ei_p, zeros2)
  f1, u1 = _tc_mid(s0a, s0b, u0, h, dinv, Wg1, bg1r, sc_alphas, ff_alphas)
  s1a, s1b = _sc_scatter_rows(u1, ei_p, zeros2)
  return _tc_post(s1a, s1b, u1, h, f1, dinv, Wc1, bc1r, Wc2, bc2r,
                  sc_alphas, ff_alphas)
